# Initial kernel scaffold; baseline (speedup 1.0000x reference)
import jax
import jax.numpy as jnp
from jax.experimental import pallas as pl


def kernel(x, edge_index, W1, b1, W2, b2):
    raise NotImplementedError("write your pallas kernel here")


# speedup vs baseline: 8.6079x; 8.6079x over previous
import functools

import jax
import jax.numpy as jnp
from jax import lax
from jax.experimental import pallas as pl
from jax.experimental.pallas import tpu as pltpu
from jax.experimental.pallas import tpu_sc as plsc

N = 10000
N_PAD = 10240
E = 320000
E_T = 10112
E_PAD = 32 * E_T
CH = 79
ROWS_T = 640
D_IN = 128
H1 = 256
H2 = 128

def _deg_body(src_hbm, dst_hbm, zvec_hbm, dout_hbm, din_hbm,
              src_v, dst_v, ones_v, dout_sh, din_sh):
    c = lax.axis_index("c")
    s = lax.axis_index("s")
    wid = c * 16 + s
    row = pl.ds(s * ROWS_T, ROWS_T)
    pltpu.sync_copy(zvec_hbm.at[row], dout_sh.at[row])
    pltpu.sync_copy(zvec_hbm.at[row], din_sh.at[row])
    pltpu.sync_copy(src_hbm.at[wid], src_v)
    pltpu.sync_copy(dst_hbm.at[wid], dst_v)
    for i in range(8):
        ones_v[pl.ds(i * 16, 16)] = jnp.full((16,), 1.0, jnp.float32)
    plsc.subcore_barrier()

    def body(j, carry):
        pltpu.sync_copy(ones_v, dout_sh.at[src_v.at[j]], add=True)
        pltpu.sync_copy(ones_v, din_sh.at[dst_v.at[j]], add=True)
        return carry

    lax.fori_loop(0, CH, body, 0)
    plsc.subcore_barrier()
    pltpu.sync_copy(dout_sh.at[row], dout_hbm.at[wid])
    pltpu.sync_copy(din_sh.at[row], din_hbm.at[wid])


def _edge_body(xp_hbm, nd_hbm, src_hbm, dst_hbm, zrow_hbm, zvec_hbm,
               agg_hbm, c_hbm,
               src_v, dst_v, rows_v, vals_v, agg_sh, c_sh, sem1, sem2):
    c = lax.axis_index("c")
    s = lax.axis_index("s")
    wid = c * 16 + s
    row = pl.ds(s * ROWS_T, ROWS_T)
    pltpu.sync_copy(zrow_hbm.at[row], agg_sh.at[row])
    pltpu.sync_copy(zvec_hbm.at[row], c_sh.at[row])
    pltpu.sync_copy(src_hbm.at[wid], src_v)
    pltpu.sync_copy(dst_hbm.at[wid], dst_v)
    plsc.subcore_barrier()

    def body(j, carry):
        g1 = pltpu.async_copy(xp_hbm.at[src_v.at[j]], rows_v, sem1)
        g2 = pltpu.async_copy(nd_hbm.at[dst_v.at[j]], vals_v, sem2)
        g1.wait()
        g2.wait()
        pltpu.sync_copy(rows_v, agg_sh.at[dst_v.at[j]], add=True)
        pltpu.sync_copy(vals_v, c_sh.at[src_v.at[j]], add=True)
        return carry

    lax.fori_loop(0, CH, body, 0)
    plsc.subcore_barrier()
    pltpu.sync_copy(agg_sh.at[row], agg_hbm.at[wid])
    pltpu.sync_copy(c_sh.at[row], c_hbm.at[wid])


@functools.cache
def _sc_kernels():
    mesh = plsc.VectorSubcoreMesh(core_axis_name="c", subcore_axis_name="s",
                                  num_cores=2, num_subcores=16)
    deg = pl.kernel(
        _deg_body,
        out_type=(
            jax.ShapeDtypeStruct((32, ROWS_T), jnp.float32),
            jax.ShapeDtypeStruct((32, ROWS_T), jnp.float32),
        ),
        mesh=mesh,
        scratch_types=[
            pltpu.VMEM((CH, 128), jnp.int32),
            pltpu.VMEM((CH, 128), jnp.int32),
            pltpu.VMEM((128,), jnp.float32),
            pltpu.VMEM_SHARED((N_PAD,), jnp.float32),
            pltpu.VMEM_SHARED((N_PAD,), jnp.float32),
        ],
    )
    edge = pl.kernel(
        _edge_body,
        out_type=(
            jax.ShapeDtypeStruct((32, ROWS_T, D_IN), jnp.float32),
            jax.ShapeDtypeStruct((32, ROWS_T), jnp.float32),
        ),
        mesh=mesh,
        scratch_types=[
            pltpu.VMEM((CH, 128), jnp.int32),
            pltpu.VMEM((CH, 128), jnp.int32),
            pltpu.VMEM((128, D_IN), jnp.float32),
            pltpu.VMEM((128,), jnp.float32),
            pltpu.VMEM_SHARED((N_PAD, D_IN), jnp.float32),
            pltpu.VMEM_SHARED((N_PAD,), jnp.float32),
            pltpu.SemaphoreType.DMA,
            pltpu.SemaphoreType.DMA,
        ],
    )
    return deg, edge


def _prep_body(x_ref, dop_ref, dip_ref, ns_ref, nd_ref, xp_ref):
    do = dop_ref[0] + dop_ref[1]
    di = dip_ref[0] + dip_ref[1]
    ns = lax.rsqrt(jnp.maximum(do, 1.0))
    nd = lax.rsqrt(jnp.maximum(di, 1.0))
    ns_ref[...] = ns
    nd_ref[...] = nd
    xp_ref[...] = x_ref[...] * ns


_prep = pl.pallas_call(
    _prep_body,
    out_shape=(
        jax.ShapeDtypeStruct((N_PAD, 1), jnp.float32),
        jax.ShapeDtypeStruct((N_PAD, 1), jnp.float32),
        jax.ShapeDtypeStruct((N_PAD, D_IN), jnp.float32),
    ),
)


def _final_body(aggp_ref, cp_ref, nd_ref, ns_ref, w1_ref, b1_ref, w2_ref,
                b2_ref, out_ref):
    agg = (aggp_ref[0] + aggp_ref[1]) * nd_ref[...]
    h = jnp.dot(agg, w1_ref[...], preferred_element_type=jnp.float32)
    q = jnp.maximum(h + b1_ref[...], 0.0) * ns_ref[...]
    cvec = cp_ref[0] + cp_ref[1]
    rid = lax.broadcasted_iota(jnp.int32, (N_PAD, 1), 0)
    cvec = jnp.where(rid < N, cvec, 0.0)
    v = jnp.sum(q * cvec, axis=0, keepdims=True)
    out_ref[...] = (jnp.dot(v, w2_ref[...], preferred_element_type=jnp.float32)
                    * (1.0 / N) + b2_ref[...])


_final = pl.pallas_call(
    _final_body,
    out_shape=jax.ShapeDtypeStruct((1, H2), jnp.float32),
)


def kernel(x, edge_index, W1, b1, W2, b2):
    src = edge_index[0].astype(jnp.int32)
    dst = edge_index[1].astype(jnp.int32)
    pad = E_PAD - E
    padv = jnp.full((pad,), N, jnp.int32)
    src_p = jnp.concatenate([src, padv]).reshape(32, CH, 128)
    dst_p = jnp.concatenate([dst, padv]).reshape(32, CH, 128)
    x_pad = jnp.zeros((N_PAD, D_IN), jnp.float32).at[:N].set(x)
    zvec = jnp.zeros((N_PAD,), jnp.float32)
    zrow = jnp.zeros((N_PAD, D_IN), jnp.float32)

    deg_kernel, edge_kernel = _sc_kernels()
    dout_t, din_t = deg_kernel(src_p, dst_p, zvec)
    ns, nd, xp = _prep(x_pad,
                       dout_t.reshape(2, N_PAD, 1),
                       din_t.reshape(2, N_PAD, 1))
    agg_t, c_t = edge_kernel(xp, nd.reshape(N_PAD), src_p, dst_p, zrow, zvec)
    out = _final(agg_t.reshape(2, N_PAD, D_IN), c_t.reshape(2, N_PAD, 1),
                 nd, ns, W1, b1.reshape(1, H1), W2, b2.reshape(1, H2))
    return out
